# Initial kernel scaffold; baseline (speedup 1.0000x reference)
import jax
import jax.numpy as jnp
from jax.experimental import pallas as pl


def kernel(x, edge_index, edge_weight, W_gcn, b_gcn, W_ih, W_hh, b_ih, b_hh, W_fc, b_fc):
    raise NotImplementedError("write your pallas kernel here")


# speedup vs baseline: 4.8762x; 4.8762x over previous
import jax
import jax.numpy as jnp
from jax.experimental import pallas as pl


def _transform_kernel(x_ref, w_ref, b_ref, o_ref):
    o_ref[...] = (
        jnp.dot(x_ref[...], w_ref[...], preferred_element_type=jnp.float32)
        + b_ref[...]
    )


def _agg_matmul_kernel(a_ref, h_ref, b_ref, o_ref):
    k = pl.program_id(1)
    part = jnp.dot(a_ref[...], h_ref[...], preferred_element_type=jnp.float32)

    @pl.when(k == 0)
    def _init():
        o_ref[...] = part + b_ref[...]

    @pl.when(k > 0)
    def _acc():
        o_ref[...] += part


def _lstm_fc_kernel(x_ref, wih_ref, whh_ref, b_ref, wfc_ref, bfc_ref, o_ref):
    nb = x_ref.shape[0]
    hdim = whh_ref.shape[0]
    wih = wih_ref[...]
    whh = whh_ref[...]
    bias = b_ref[...]
    h = jnp.zeros((nb, hdim), dtype=jnp.float32)
    c = jnp.zeros((nb, hdim), dtype=jnp.float32)
    for t in range(x_ref.shape[1]):
        xt = x_ref[:, t, :]
        gates = (
            jnp.dot(xt, wih, preferred_element_type=jnp.float32)
            + jnp.dot(h, whh, preferred_element_type=jnp.float32)
            + bias
        )
        i = jax.nn.sigmoid(gates[:, :hdim])
        f = jax.nn.sigmoid(gates[:, hdim : 2 * hdim])
        g = jnp.tanh(gates[:, 2 * hdim : 3 * hdim])
        o = jax.nn.sigmoid(gates[:, 3 * hdim :])
        c = f * c + i * g
        h = o * jnp.tanh(c)
    o_ref[...] = (
        jnp.dot(h, wfc_ref[...], preferred_element_type=jnp.float32) + bfc_ref[...]
    )


def kernel(x, edge_index, edge_weight, W_gcn, b_gcn, W_ih, W_hh, b_ih, b_hh, W_fc, b_fc):
    B, S, N, F = x.shape
    H = W_gcn.shape[0]
    O = W_fc.shape[0]
    BS = B * S
    E = edge_weight.shape[0]

    row, col = edge_index[0], edge_index[1]
    loop = jnp.arange(N, dtype=row.dtype)
    row2 = jnp.concatenate([row, loop])
    col2 = jnp.concatenate([col, loop])
    ew2 = jnp.concatenate([edge_weight, jnp.ones((N,), dtype=edge_weight.dtype)])
    deg = jnp.zeros((N,), dtype=jnp.float32).at[col2].add(ew2)
    dinv = jnp.where(deg > 0, jax.lax.rsqrt(deg), 0.0)
    norm = dinv[row2] * ew2 * dinv[col2]

    NP = 5120 if N == 5000 else ((N + 127) // 128) * 128
    flat_idx = col2.astype(jnp.int32) * NP + row2.astype(jnp.int32)
    A = (
        jnp.zeros((NP * NP,), dtype=jnp.float32)
        .at[flat_idx]
        .add(norm)
        .reshape(NP, NP)
    )

    xf = x.reshape(BS * N, F)
    rows = BS * N
    bm1 = 2000
    grid1 = rows // bm1
    W_T = W_gcn.T
    h = pl.pallas_call(
        _transform_kernel,
        grid=(grid1,),
        in_specs=[
            pl.BlockSpec((bm1, F), lambda i: (i, 0)),
            pl.BlockSpec((F, H), lambda i: (0, 0)),
            pl.BlockSpec((1, H), lambda i: (0, 0)),
        ],
        out_specs=pl.BlockSpec((bm1, H), lambda i: (i, 0)),
        out_shape=jax.ShapeDtypeStruct((rows, H), jnp.float32),
    )(xf, W_T, jnp.zeros((1, H), jnp.float32))

    hfeat = h.reshape(BS, N, H).transpose(1, 0, 2).reshape(N, BS * H)
    hfeat = jnp.pad(hfeat, ((0, NP - N), (0, 0)))

    bm2, bk2 = 512, 1024
    cols = BS * H
    b_tiled = jnp.tile(b_gcn, BS).reshape(1, cols)
    agg = pl.pallas_call(
        _agg_matmul_kernel,
        grid=(NP // bm2, NP // bk2),
        in_specs=[
            pl.BlockSpec((bm2, bk2), lambda i, k: (i, k)),
            pl.BlockSpec((bk2, cols), lambda i, k: (k, 0)),
            pl.BlockSpec((1, cols), lambda i, k: (0, 0)),
        ],
        out_specs=pl.BlockSpec((bm2, cols), lambda i, k: (i, 0)),
        out_shape=jax.ShapeDtypeStruct((NP, cols), jnp.float32),
    )(A, hfeat, b_tiled)

    seq = agg[:N].reshape(N, B, S, H).transpose(1, 0, 2, 3).reshape(B * N, S, H)
    BN = B * N
    bm3 = 512
    BNP = ((BN + bm3 - 1) // bm3) * bm3
    seq = jnp.pad(seq, ((0, BNP - BN), (0, 0), (0, 0)))

    out = pl.pallas_call(
        _lstm_fc_kernel,
        grid=(BNP // bm3,),
        in_specs=[
            pl.BlockSpec((bm3, S, H), lambda i: (i, 0, 0)),
            pl.BlockSpec((H, 4 * H), lambda i: (0, 0)),
            pl.BlockSpec((H, 4 * H), lambda i: (0, 0)),
            pl.BlockSpec((1, 4 * H), lambda i: (0, 0)),
            pl.BlockSpec((H, O), lambda i: (0, 0)),
            pl.BlockSpec((1, O), lambda i: (0, 0)),
        ],
        out_specs=pl.BlockSpec((bm3, O), lambda i: (i, 0)),
        out_shape=jax.ShapeDtypeStruct((BNP, O), jnp.float32),
    )(
        seq,
        W_ih.T,
        W_hh.T,
        (b_ih + b_hh).reshape(1, 4 * H),
        W_fc.T,
        b_fc.reshape(1, O),
    )

    return out[:BN].reshape(B, N, O)


# speedup vs baseline: 5.2709x; 1.0809x over previous
import jax
import jax.numpy as jnp
from jax.experimental import pallas as pl


def _transform_kernel(x_ref, w_ref, b_ref, o_ref):
    o_ref[...] = (
        jnp.dot(x_ref[...], w_ref[...], preferred_element_type=jnp.float32)
        + b_ref[...]
    ).astype(o_ref.dtype)


def _agg_matmul_kernel(a_ref, h_ref, b_ref, o_ref):
    k = pl.program_id(1)
    part = jnp.dot(a_ref[...], h_ref[...], preferred_element_type=jnp.float32)

    @pl.when(k == 0)
    def _init():
        o_ref[...] = part + b_ref[...]

    @pl.when(k > 0)
    def _acc():
        o_ref[...] += part


def _lstm_fc_kernel(x_ref, wih_ref, whh_ref, b_ref, wfc_ref, bfc_ref, o_ref):
    nb = x_ref.shape[0]
    hdim = whh_ref.shape[0]
    wih = wih_ref[...]
    whh = whh_ref[...]
    bias = b_ref[...]
    h = jnp.zeros((nb, hdim), dtype=jnp.float32)
    c = jnp.zeros((nb, hdim), dtype=jnp.float32)
    for t in range(x_ref.shape[1]):
        xt = x_ref[:, t, :]
        gates = (
            jnp.dot(xt, wih, preferred_element_type=jnp.float32)
            + jnp.dot(h, whh, preferred_element_type=jnp.float32)
            + bias
        )
        i = jax.nn.sigmoid(gates[:, :hdim])
        f = jax.nn.sigmoid(gates[:, hdim : 2 * hdim])
        g = jnp.tanh(gates[:, 2 * hdim : 3 * hdim])
        o = jax.nn.sigmoid(gates[:, 3 * hdim :])
        c = f * c + i * g
        h = o * jnp.tanh(c)
    o_ref[...] = (
        jnp.dot(h, wfc_ref[...], preferred_element_type=jnp.float32) + bfc_ref[...]
    )


def kernel(x, edge_index, edge_weight, W_gcn, b_gcn, W_ih, W_hh, b_ih, b_hh, W_fc, b_fc):
    B, S, N, F = x.shape
    H = W_gcn.shape[0]
    O = W_fc.shape[0]
    BS = B * S
    E = edge_weight.shape[0]

    row, col = edge_index[0], edge_index[1]
    loop = jnp.arange(N, dtype=row.dtype)
    row2 = jnp.concatenate([row, loop])
    col2 = jnp.concatenate([col, loop])
    ew2 = jnp.concatenate([edge_weight, jnp.ones((N,), dtype=edge_weight.dtype)])
    deg = jnp.zeros((N,), dtype=jnp.float32).at[col2].add(ew2)
    dinv = jnp.where(deg > 0, jax.lax.rsqrt(deg), 0.0)
    norm = dinv[row2] * ew2 * dinv[col2]

    NP = 5120 if N == 5000 else ((N + 127) // 128) * 128
    flat_idx = col2.astype(jnp.int32) * NP + row2.astype(jnp.int32)
    A = (
        jnp.zeros((NP * NP,), dtype=jnp.float32)
        .at[flat_idx]
        .add(norm)
        .reshape(NP, NP)
    ).astype(jnp.bfloat16)

    xf = x.reshape(BS * N, F)
    rows = BS * N
    bm1 = 2000
    grid1 = rows // bm1
    W_T = W_gcn.T
    h = pl.pallas_call(
        _transform_kernel,
        grid=(grid1,),
        in_specs=[
            pl.BlockSpec((bm1, F), lambda i: (i, 0)),
            pl.BlockSpec((F, H), lambda i: (0, 0)),
            pl.BlockSpec((1, H), lambda i: (0, 0)),
        ],
        out_specs=pl.BlockSpec((bm1, H), lambda i: (i, 0)),
        out_shape=jax.ShapeDtypeStruct((rows, H), jnp.bfloat16),
    )(xf, W_T, jnp.zeros((1, H), jnp.float32))

    hfeat = h.reshape(BS, N, H).transpose(1, 0, 2).reshape(N, BS * H)
    hfeat = jnp.pad(hfeat, ((0, NP - N), (0, 0)))

    bm2, bk2 = 512, 1024
    cols = BS * H
    b_tiled = jnp.tile(b_gcn, BS).reshape(1, cols)
    agg = pl.pallas_call(
        _agg_matmul_kernel,
        grid=(NP // bm2, NP // bk2),
        in_specs=[
            pl.BlockSpec((bm2, bk2), lambda i, k: (i, k)),
            pl.BlockSpec((bk2, cols), lambda i, k: (k, 0)),
            pl.BlockSpec((1, cols), lambda i, k: (0, 0)),
        ],
        out_specs=pl.BlockSpec((bm2, cols), lambda i, k: (i, 0)),
        out_shape=jax.ShapeDtypeStruct((NP, cols), jnp.float32),
    )(A, hfeat, b_tiled)

    seq = agg[:N].reshape(N, B, S, H).transpose(1, 0, 2, 3).reshape(B * N, S, H)
    BN = B * N
    bm3 = 512
    BNP = ((BN + bm3 - 1) // bm3) * bm3
    seq = jnp.pad(seq, ((0, BNP - BN), (0, 0), (0, 0)))

    out = pl.pallas_call(
        _lstm_fc_kernel,
        grid=(BNP // bm3,),
        in_specs=[
            pl.BlockSpec((bm3, S, H), lambda i: (i, 0, 0)),
            pl.BlockSpec((H, 4 * H), lambda i: (0, 0)),
            pl.BlockSpec((H, 4 * H), lambda i: (0, 0)),
            pl.BlockSpec((1, 4 * H), lambda i: (0, 0)),
            pl.BlockSpec((H, O), lambda i: (0, 0)),
            pl.BlockSpec((1, O), lambda i: (0, 0)),
        ],
        out_specs=pl.BlockSpec((bm3, O), lambda i: (i, 0)),
        out_shape=jax.ShapeDtypeStruct((BNP, O), jnp.float32),
    )(
        seq,
        W_ih.T,
        W_hh.T,
        (b_ih + b_hh).reshape(1, 4 * H),
        W_fc.T,
        b_fc.reshape(1, O),
    )

    return out[:BN].reshape(B, N, O)


# speedup vs baseline: 5.2727x; 1.0003x over previous
import jax
import jax.numpy as jnp
from jax.experimental import pallas as pl


def _transform_kernel(x_ref, w_ref, b_ref, o_ref):
    o_ref[...] = (
        jnp.dot(x_ref[...], w_ref[...], preferred_element_type=jnp.float32)
        + b_ref[...]
    ).astype(o_ref.dtype)


def _agg_matmul_kernel(a_ref, h_ref, b_ref, o_ref):
    k = pl.program_id(1)
    part = jnp.dot(a_ref[...], h_ref[...], preferred_element_type=jnp.float32)

    @pl.when(k == 0)
    def _init():
        o_ref[...] = part + b_ref[...]

    @pl.when(k > 0)
    def _acc():
        o_ref[...] += part


def _make_lstm_fc_kernel(nbatch, nsteps):
    def _lstm_fc_kernel(x_ref, wih_ref, whh_ref, b_ref, wfc_ref, bfc_ref, o_ref):
        nb = x_ref.shape[0]
        hdim = whh_ref.shape[0]
        wih = wih_ref[...]
        whh = whh_ref[...]
        bias = b_ref[...]
        wfc = wfc_ref[...]
        bfc = bfc_ref[...]
        for b in range(nbatch):
            h = jnp.zeros((nb, hdim), dtype=jnp.float32)
            c = jnp.zeros((nb, hdim), dtype=jnp.float32)
            for t in range(nsteps):
                s = (b * nsteps + t) * hdim
                xt = x_ref[:, s : s + hdim]
                gates = (
                    jnp.dot(xt, wih, preferred_element_type=jnp.float32)
                    + jnp.dot(h, whh, preferred_element_type=jnp.float32)
                    + bias
                )
                i = jax.nn.sigmoid(gates[:, :hdim])
                f = jax.nn.sigmoid(gates[:, hdim : 2 * hdim])
                g = jnp.tanh(gates[:, 2 * hdim : 3 * hdim])
                o = jax.nn.sigmoid(gates[:, 3 * hdim :])
                c = f * c + i * g
                h = o * jnp.tanh(c)
            o_ref[:, b, :] = (
                jnp.dot(h, wfc, preferred_element_type=jnp.float32) + bfc
            )

    return _lstm_fc_kernel


def kernel(x, edge_index, edge_weight, W_gcn, b_gcn, W_ih, W_hh, b_ih, b_hh, W_fc, b_fc):
    B, S, N, F = x.shape
    H = W_gcn.shape[0]
    O = W_fc.shape[0]
    BS = B * S
    E = edge_weight.shape[0]

    row, col = edge_index[0], edge_index[1]
    loop = jnp.arange(N, dtype=row.dtype)
    row2 = jnp.concatenate([row, loop])
    col2 = jnp.concatenate([col, loop])
    ew2 = jnp.concatenate([edge_weight, jnp.ones((N,), dtype=edge_weight.dtype)])
    deg = jnp.zeros((N,), dtype=jnp.float32).at[col2].add(ew2)
    dinv = jnp.where(deg > 0, jax.lax.rsqrt(deg), 0.0)
    norm = dinv[row2] * ew2 * dinv[col2]

    NP = 5120 if N == 5000 else ((N + 127) // 128) * 128
    flat_idx = col2.astype(jnp.int32) * NP + row2.astype(jnp.int32)
    A = (
        jnp.zeros((NP * NP,), dtype=jnp.bfloat16)
        .at[flat_idx]
        .add(norm.astype(jnp.bfloat16))
        .reshape(NP, NP)
    )

    xf = x.reshape(BS * N, F)
    rows = BS * N
    bm1 = 2000
    grid1 = rows // bm1
    W_T = W_gcn.T
    h = pl.pallas_call(
        _transform_kernel,
        grid=(grid1,),
        in_specs=[
            pl.BlockSpec((bm1, F), lambda i: (i, 0)),
            pl.BlockSpec((F, H), lambda i: (0, 0)),
            pl.BlockSpec((1, H), lambda i: (0, 0)),
        ],
        out_specs=pl.BlockSpec((bm1, H), lambda i: (i, 0)),
        out_shape=jax.ShapeDtypeStruct((rows, H), jnp.bfloat16),
    )(xf, W_T, jnp.zeros((1, H), jnp.float32))

    hfeat = h.reshape(BS, N, H).transpose(1, 0, 2).reshape(N, BS * H)
    hfeat = jnp.pad(hfeat, ((0, NP - N), (0, 0)))

    bm2, bk2 = 512, 1024
    cols = BS * H
    b_tiled = jnp.tile(b_gcn, BS).reshape(1, cols)
    agg = pl.pallas_call(
        _agg_matmul_kernel,
        grid=(NP // bm2, NP // bk2),
        in_specs=[
            pl.BlockSpec((bm2, bk2), lambda i, k: (i, k)),
            pl.BlockSpec((bk2, cols), lambda i, k: (k, 0)),
            pl.BlockSpec((1, cols), lambda i, k: (0, 0)),
        ],
        out_specs=pl.BlockSpec((bm2, cols), lambda i, k: (i, 0)),
        out_shape=jax.ShapeDtypeStruct((NP, cols), jnp.float32),
    )(A, hfeat, b_tiled)

    bm3 = 512
    out = pl.pallas_call(
        _make_lstm_fc_kernel(B, S),
        grid=(NP // bm3,),
        in_specs=[
            pl.BlockSpec((bm3, cols), lambda i: (i, 0)),
            pl.BlockSpec((H, 4 * H), lambda i: (0, 0)),
            pl.BlockSpec((H, 4 * H), lambda i: (0, 0)),
            pl.BlockSpec((1, 4 * H), lambda i: (0, 0)),
            pl.BlockSpec((H, O), lambda i: (0, 0)),
            pl.BlockSpec((1, O), lambda i: (0, 0)),
        ],
        out_specs=pl.BlockSpec((bm3, B, O), lambda i: (i, 0, 0)),
        out_shape=jax.ShapeDtypeStruct((NP, B, O), jnp.float32),
    )(
        agg,
        W_ih.T,
        W_hh.T,
        (b_ih + b_hh).reshape(1, 4 * H),
        W_fc.T,
        b_fc.reshape(1, O),
    )

    return out[:N].transpose(1, 0, 2)
